# Initial kernel scaffold; baseline (speedup 1.0000x reference)
import jax
import jax.numpy as jnp
from jax.experimental import pallas as pl


def kernel(x, edge_index, edge_attr, batch, c0_pre0_W, c0_pre0_b, c0_pre1_W, c0_pre1_b, c0_g0_W, c0_g0_b, c0_g1_W, c0_g1_b, c1_pre0_W, c1_pre0_b, c1_pre1_W, c1_pre1_b, c1_g0_W, c1_g0_b, c1_g1_W, c1_g1_b, cls_W, cls_b):
    raise NotImplementedError("write your pallas kernel here")


# speedup vs baseline: 12.7409x; 12.7409x over previous
import functools

import jax
import jax.numpy as jnp
from jax import lax
from jax.experimental import pallas as pl
from jax.experimental.pallas import tpu as pltpu
from jax.experimental.pallas import tpu_sc as plsc

N = 10000
D = 128
G = 16
NC = 2
NS = 16
NW = NC * NS
CH = 128
NPAD = 10240
RPS = NPAD // NS
BR = 1000
PREC = lax.Precision.HIGHEST

_mesh = plsc.VectorSubcoreMesh(core_axis_name="c", subcore_axis_name="s")


def _sc_deg_body(dst_hbm, ones_hbm, zeros_hbm, out_hbm, dstv, onesv, acc, sem):
    c = lax.axis_index("c")
    s = lax.axis_index("s")
    w = c * NS + s
    cpt = dst_hbm.shape[0] // NW
    pltpu.sync_copy(zeros_hbm, acc.at[pl.ds(s * RPS, RPS)])
    pltpu.sync_copy(dst_hbm.at[pl.ds(w * cpt, cpt)], dstv)
    pltpu.sync_copy(ones_hbm, onesv)
    plsc.subcore_barrier()

    def body(j, carry):
        pltpu.sync_copy(onesv, acc.at[dstv.at[j]], add=True)
        return carry

    lax.fori_loop(0, cpt, body, 0)
    plsc.subcore_barrier()
    pltpu.sync_copy(acc.at[pl.ds(s * RPS, RPS)],
                    out_hbm.at[c, pl.ds(s * RPS, RPS)])


def _sc_apply_body(mp_hbm, src_hbm, dst_hbm, zeros_hbm, out_hbm,
                   srcv, dstv, rowsv, acc, sem):
    c = lax.axis_index("c")
    s = lax.axis_index("s")
    w = c * NS + s
    cpt = src_hbm.shape[0] // NW
    pltpu.sync_copy(zeros_hbm, acc.at[pl.ds(s * RPS, RPS)])
    pltpu.sync_copy(src_hbm.at[pl.ds(w * cpt, cpt)], srcv)
    pltpu.sync_copy(dst_hbm.at[pl.ds(w * cpt, cpt)], dstv)
    plsc.subcore_barrier()

    def body(j, carry):
        pltpu.async_copy(mp_hbm.at[srcv.at[j]], rowsv, sem).wait()
        pltpu.sync_copy(rowsv, acc.at[dstv.at[j]], add=True)
        return carry

    lax.fori_loop(0, cpt, body, 0)
    plsc.subcore_barrier()
    pltpu.sync_copy(acc.at[pl.ds(s * RPS, RPS)],
                    out_hbm.at[c, pl.ds(s * RPS, RPS)])


def _make_sc_deg(cpt):
    return functools.partial(
        pl.kernel,
        out_type=jax.ShapeDtypeStruct((NC, NPAD, D), jnp.float32),
        mesh=_mesh,
        scratch_types=[
            pltpu.VMEM((cpt, CH), jnp.int32),
            pltpu.VMEM((CH, D), jnp.float32),
            pltpu.VMEM_SHARED((NPAD, D), jnp.float32),
            pltpu.SemaphoreType.DMA,
        ],
    )(_sc_deg_body)


def _make_sc_apply(cpt):
    return functools.partial(
        pl.kernel,
        out_type=jax.ShapeDtypeStruct((NC, NPAD, D), jnp.float32),
        mesh=_mesh,
        scratch_types=[
            pltpu.VMEM((cpt, CH), jnp.int32),
            pltpu.VMEM((cpt, CH), jnp.int32),
            pltpu.VMEM((CH, D), jnp.float32),
            pltpu.VMEM_SHARED((NPAD, D), jnp.float32),
            pltpu.SemaphoreType.DMA,
        ],
    )(_sc_apply_body)


def _dinv_from(degp_ref):
    degsum = degp_ref[0, :, 0:1] + degp_ref[1, :, 0:1]
    deg = degsum + 1.0
    return lax.rsqrt(jnp.maximum(deg, 1.0))


def _tk1_body(x_ref, wa_ref, bpre_ref, bias_ref, degp_ref, mp_ref, base_ref):
    dinv = _dinv_from(degp_ref)
    mu = jnp.dot(x_ref[...], wa_ref[...],
                 preferred_element_type=jnp.float32, precision=PREC)
    mu = mu + bpre_ref[...]
    mp_ref[...] = dinv * mu
    base_ref[...] = (dinv * dinv) * mu + bias_ref[...]


def _tk2_body(x_ref, r_ref, basea_ref, wb1_ref, wb2_ref, bpre_ref, bias_ref,
              degp_ref, mp_ref, base_ref):
    dinv = _dinv_from(degp_ref)
    a = dinv * (r_ref[0] + r_ref[1]) + basea_ref[...]
    mu = (jnp.dot(x_ref[...], wb1_ref[...],
                  preferred_element_type=jnp.float32, precision=PREC)
          + jnp.dot(a, wb2_ref[...],
                    preferred_element_type=jnp.float32, precision=PREC)
          + bpre_ref[...])
    mp_ref[...] = dinv * mu
    base_ref[...] = (dinv * dinv) * mu + bias_ref[...]


def _tk3_body(r_ref, baseb_ref, degp_ref, oneh_ref, clsw_ref, clsb_ref,
              out_ref, sums, maxs, cnts):
    i = pl.program_id(0)

    @pl.when(i == 0)
    def _init():
        sums[...] = jnp.zeros_like(sums)
        maxs[...] = jnp.full_like(maxs, -jnp.inf)
        cnts[...] = jnp.zeros_like(cnts)

    dinv = _dinv_from(degp_ref)
    b = dinv * (r_ref[0] + r_ref[1]) + baseb_ref[...]
    oh = oneh_ref[...]
    dn = (((0,), (0,)), ((), ()))
    sums[...] += lax.dot_general(oh, b, dn,
                                 preferred_element_type=jnp.float32,
                                 precision=PREC)
    cnts[...] += lax.dot_general(oh, jnp.ones_like(b), dn,
                                 preferred_element_type=jnp.float32,
                                 precision=PREC)
    for g in range(G):
        col = oh[:, g:g + 1]
        masked = jnp.where(col > 0.5, b, -jnp.inf)
        mg = jnp.max(masked, axis=0, keepdims=True)
        maxs[g:g + 1, :] = jnp.maximum(maxs[g:g + 1, :], mg)

    @pl.when(i == pl.num_programs(0) - 1)
    def _fin():
        mean = sums[...] / jnp.maximum(cnts[...], 1.0)
        pooled = mean + maxs[...]
        out_ref[...] = (jnp.dot(pooled, clsw_ref[...],
                                preferred_element_type=jnp.float32,
                                precision=PREC)
                        + clsb_ref[...])


def _row_spec():
    return pl.BlockSpec((BR, D), lambda i: (i, 0))


def _full_spec(shape):
    nd = len(shape)
    return pl.BlockSpec(shape, lambda i, _n=nd: (0,) * _n)


def _degp_spec():
    return pl.BlockSpec((NC, BR, D), lambda i: (0, i, 0))


def _r_spec():
    return pl.BlockSpec((NC, BR, D), lambda i: (0, i, 0))


_GRID = (N // BR,)

_tk1 = pl.pallas_call(
    _tk1_body,
    grid=_GRID,
    in_specs=[_row_spec(), _full_spec((D, D)), _full_spec((1, D)),
              _full_spec((1, D)), _degp_spec()],
    out_specs=[_row_spec(), _row_spec()],
    out_shape=[jax.ShapeDtypeStruct((N, D), jnp.float32),
               jax.ShapeDtypeStruct((N, D), jnp.float32)],
)

_tk2 = pl.pallas_call(
    _tk2_body,
    grid=_GRID,
    in_specs=[_row_spec(), _r_spec(), _row_spec(), _full_spec((D, D)),
              _full_spec((D, D)), _full_spec((1, D)), _full_spec((1, D)),
              _degp_spec()],
    out_specs=[_row_spec(), _row_spec()],
    out_shape=[jax.ShapeDtypeStruct((N, D), jnp.float32),
               jax.ShapeDtypeStruct((N, D), jnp.float32)],
)

_tk3 = pl.pallas_call(
    _tk3_body,
    grid=_GRID,
    in_specs=[_r_spec(), _row_spec(), _degp_spec(),
              pl.BlockSpec((BR, G), lambda i: (i, 0)),
              _full_spec((D, 10)), _full_spec((1, 10))],
    out_specs=pl.BlockSpec((G, 10), lambda i: (0, 0)),
    out_shape=jax.ShapeDtypeStruct((G, 10), jnp.float32),
    scratch_shapes=[pltpu.VMEM((G, D), jnp.float32),
                    pltpu.VMEM((G, D), jnp.float32),
                    pltpu.VMEM((G, D), jnp.float32)],
)


def kernel(x, edge_index, edge_attr, batch,
           c0_pre0_W, c0_pre0_b, c0_pre1_W, c0_pre1_b,
           c0_g0_W, c0_g0_b, c0_g1_W, c0_g1_b,
           c1_pre0_W, c1_pre0_b, c1_pre1_W, c1_pre1_b,
           c1_g0_W, c1_g0_b, c1_g1_W, c1_g1_b,
           cls_W, cls_b):
    f32 = jnp.float32
    e = edge_index.shape[1]
    pt = e // NW
    cpt = -(-(-(-pt // CH)) // 8) * 8
    ptp = cpt * CH

    src = edge_index[0].reshape(NW, pt)
    dst = edge_index[1].reshape(NW, pt)
    pad = ptp - pt
    src_pad = jnp.concatenate(
        [src, jnp.zeros((NW, pad), jnp.int32)], axis=1).reshape(NW * cpt, CH)
    dst_pad = jnp.concatenate(
        [dst, jnp.full((NW, pad), N, jnp.int32)], axis=1).reshape(NW * cpt, CH)

    def mm(a, b):
        return jnp.dot(a, b, preferred_element_type=f32, precision=PREC)

    w_a = 2.0 * mm(c0_pre0_W, c0_g0_W) + mm(c0_pre1_W, c0_g1_W)
    bpre_a = 2.0 * mm(c0_pre0_b[None], c0_g0_W) + mm(c0_pre1_b[None], c0_g1_W)
    bias_a = (2.0 * c0_g0_b + c0_g1_b)[None]
    w_b1 = 2.0 * mm(c1_pre0_W, c1_g0_W)
    w_b2 = mm(c1_pre1_W, c1_g1_W)
    bpre_b = 2.0 * mm(c1_pre0_b[None], c1_g0_W) + mm(c1_pre1_b[None], c1_g1_W)
    bias_b = (2.0 * c1_g0_b + c1_g1_b)[None]

    ones128 = jnp.ones((CH, D), f32)
    zeros128 = jnp.zeros((RPS, D), f32)

    sc_deg = _make_sc_deg(cpt)
    sc_apply = _make_sc_apply(cpt)

    degp = sc_deg(dst_pad, ones128, zeros128)
    mp_a, base_a = _tk1(x, w_a, bpre_a, bias_a, degp)
    r_a = sc_apply(mp_a, src_pad, dst_pad, zeros128)
    mp_b, base_b = _tk2(x, r_a, base_a, w_b1, w_b2, bpre_b, bias_b, degp)
    r_b = sc_apply(mp_b, src_pad, dst_pad, zeros128)

    oneh = (batch[:, None] == jnp.arange(G, dtype=batch.dtype)).astype(f32)
    scores = _tk3(r_b, base_b, degp, oneh, cls_W, cls_b[None])
    return scores


# speedup vs baseline: 20.1231x; 1.5794x over previous
import functools

import jax
import jax.numpy as jnp
from jax import lax
from jax.experimental import pallas as pl
from jax.experimental.pallas import tpu as pltpu
from jax.experimental.pallas import tpu_sc as plsc

N = 10000
D = 128
G = 16
NC = 2
NS = 16
NW = NC * NS
CH = 128
NPAD = 10112
RPS = NPAD // NS
BR = 1000
PREC = lax.Precision.HIGHEST
RING = 2
RI = 4
DEGQ = 6

_mesh = plsc.VectorSubcoreMesh(core_axis_name="c", subcore_axis_name="s")


def _sc_deg_body(dst_hbm, ones_hbm, zeros_hbm, out_hbm, dstv, onesv, acc, sem):
    c = lax.axis_index("c")
    s = lax.axis_index("s")
    w = c * NS + s
    cpt = dst_hbm.shape[0] // NW
    pltpu.sync_copy(zeros_hbm, acc.at[pl.ds(s * RPS, RPS)])
    pltpu.sync_copy(dst_hbm.at[pl.ds(w * cpt, cpt)], dstv)
    pltpu.sync_copy(ones_hbm, onesv)
    plsc.subcore_barrier()

    def body(j, carry):
        pltpu.async_copy(onesv, acc.at[dstv.at[j, 0]], sem, add=True)

        @pl.when(j >= DEGQ)
        def _():
            pltpu.make_async_copy(ones_hbm, onesv, sem).wait()
        return carry

    lax.fori_loop(0, cpt, body, 0)
    for _ in range(DEGQ):
        pltpu.make_async_copy(ones_hbm, onesv, sem).wait()
    plsc.subcore_barrier()
    pltpu.sync_copy(acc.at[pl.ds(s * RPS, RPS)],
                    out_hbm.at[c, pl.ds(s * RPS, RPS)])


def _sc_apply_body(mp_hbm, src_hbm, dst_hbm, zeros_hbm, out_hbm,
                   srcv, dstv, rowsv, acc, gsem, ssem, isem):
    c = lax.axis_index("c")
    s = lax.axis_index("s")
    w = c * NS + s
    cpt = src_hbm.shape[0] // NW
    base = w * cpt
    pltpu.sync_copy(zeros_hbm, acc.at[pl.ds(s * RPS, RPS)])
    plsc.subcore_barrier()

    def load_idx(j):
        slot = lax.rem(j, RI)
        pltpu.async_copy(src_hbm.at[base + j], srcv.at[slot], isem)
        pltpu.async_copy(dst_hbm.at[base + j], dstv.at[slot], isem)

    def drain_idx_pair():
        pltpu.make_async_copy(src_hbm.at[base], srcv.at[0], isem).wait()
        pltpu.make_async_copy(src_hbm.at[base], dstv.at[0], isem).wait()

    def gather(j):
        pltpu.async_copy(mp_hbm.at[srcv.at[lax.rem(j, RI), 0]],
                         rowsv.at[lax.rem(j, RING)], gsem)

    def wait_gather():
        pltpu.make_async_copy(mp_hbm.at[pl.ds(0, CH)],
                              rowsv.at[0], gsem).wait()

    def scatter(j):
        pltpu.async_copy(rowsv.at[lax.rem(j, RING)],
                         acc.at[dstv.at[lax.rem(j, RI), 0]], ssem, add=True)

    def drain_scatter():
        pltpu.make_async_copy(mp_hbm.at[pl.ds(0, CH)],
                              rowsv.at[0], ssem).wait()

    load_idx(0)
    load_idx(1)
    drain_idx_pair()
    gather(0)

    def body(j, carry):
        @pl.when(j + 2 < cpt)
        def _():
            load_idx(j + 2)
        wait_gather()

        scatter(j)

        @pl.when(j >= 1)
        def _():
            drain_scatter()

        @pl.when(j + 1 < cpt)
        def _():
            drain_idx_pair()
            gather(j + 1)
        return carry

    lax.fori_loop(0, cpt, body, 0)
    drain_scatter()
    plsc.subcore_barrier()
    pltpu.sync_copy(acc.at[pl.ds(s * RPS, RPS)],
                    out_hbm.at[c, pl.ds(s * RPS, RPS)])


def _make_sc_deg(cpt):
    return functools.partial(
        pl.kernel,
        out_type=jax.ShapeDtypeStruct((NC, NPAD, D), jnp.float32),
        mesh=_mesh,
        scratch_types=[
            pltpu.VMEM((cpt, 1, CH), jnp.int32),
            pltpu.VMEM((CH, D), jnp.float32),
            pltpu.VMEM_SHARED((NPAD, D), jnp.float32),
            pltpu.SemaphoreType.DMA,
        ],
    )(_sc_deg_body)


def _make_sc_apply(cpt):
    return functools.partial(
        pl.kernel,
        out_type=jax.ShapeDtypeStruct((NC, NPAD, D), jnp.float32),
        mesh=_mesh,
        scratch_types=[
            pltpu.VMEM((RI, 1, CH), jnp.int32),
            pltpu.VMEM((RI, 1, CH), jnp.int32),
            pltpu.VMEM((RING, CH, D), jnp.float32),
            pltpu.VMEM_SHARED((NPAD, D), jnp.float32),
            pltpu.SemaphoreType.DMA,
            pltpu.SemaphoreType.DMA,
            pltpu.SemaphoreType.DMA,
        ],
    )(_sc_apply_body)


def _dinv_from(degp_ref):
    degsum = degp_ref[0, :, 0:1] + degp_ref[1, :, 0:1]
    deg = degsum + 1.0
    return lax.rsqrt(jnp.maximum(deg, 1.0))


def _tk1_body(x_ref, wa_ref, bpre_ref, bias_ref, degp_ref, mp_ref, base_ref):
    dinv = _dinv_from(degp_ref)
    mu = jnp.dot(x_ref[...], wa_ref[...],
                 preferred_element_type=jnp.float32, precision=PREC)
    mu = mu + bpre_ref[...]
    mp_ref[...] = dinv * mu
    base_ref[...] = (dinv * dinv) * mu + bias_ref[...]


def _tk2_body(x_ref, r_ref, basea_ref, wb1_ref, wb2_ref, bpre_ref, bias_ref,
              degp_ref, mp_ref, base_ref):
    dinv = _dinv_from(degp_ref)
    a = dinv * (r_ref[0] + r_ref[1]) + basea_ref[...]
    mu = (jnp.dot(x_ref[...], wb1_ref[...],
                  preferred_element_type=jnp.float32, precision=PREC)
          + jnp.dot(a, wb2_ref[...],
                    preferred_element_type=jnp.float32, precision=PREC)
          + bpre_ref[...])
    mp_ref[...] = dinv * mu
    base_ref[...] = (dinv * dinv) * mu + bias_ref[...]


def _tk3_body(r_ref, baseb_ref, degp_ref, oneh_ref, clsw_ref, clsb_ref,
              out_ref, sums, maxs, cnts):
    i = pl.program_id(0)

    @pl.when(i == 0)
    def _init():
        sums[...] = jnp.zeros_like(sums)
        maxs[...] = jnp.full_like(maxs, -jnp.inf)
        cnts[...] = jnp.zeros_like(cnts)

    dinv = _dinv_from(degp_ref)
    b = dinv * (r_ref[0] + r_ref[1]) + baseb_ref[...]
    oh = oneh_ref[...]
    dn = (((0,), (0,)), ((), ()))
    sums[...] += lax.dot_general(oh, b, dn,
                                 preferred_element_type=jnp.float32,
                                 precision=PREC)
    cnts[...] += lax.dot_general(oh, jnp.ones_like(b), dn,
                                 preferred_element_type=jnp.float32,
                                 precision=PREC)
    for g in range(G):
        col = oh[:, g:g + 1]
        masked = jnp.where(col > 0.5, b, -jnp.inf)
        mg = jnp.max(masked, axis=0, keepdims=True)
        maxs[g:g + 1, :] = jnp.maximum(maxs[g:g + 1, :], mg)

    @pl.when(i == pl.num_programs(0) - 1)
    def _fin():
        mean = sums[...] / jnp.maximum(cnts[...], 1.0)
        pooled = mean + maxs[...]
        out_ref[...] = (jnp.dot(pooled, clsw_ref[...],
                                preferred_element_type=jnp.float32,
                                precision=PREC)
                        + clsb_ref[...])


def _row_spec():
    return pl.BlockSpec((BR, D), lambda i: (i, 0))


def _full_spec(shape):
    nd = len(shape)
    return pl.BlockSpec(shape, lambda i, _n=nd: (0,) * _n)


def _degp_spec():
    return pl.BlockSpec((NC, BR, D), lambda i: (0, i, 0))


def _r_spec():
    return pl.BlockSpec((NC, BR, D), lambda i: (0, i, 0))


_GRID = (N // BR,)

_tk1 = pl.pallas_call(
    _tk1_body,
    grid=_GRID,
    in_specs=[_row_spec(), _full_spec((D, D)), _full_spec((1, D)),
              _full_spec((1, D)), _degp_spec()],
    out_specs=[_row_spec(), _row_spec()],
    out_shape=[jax.ShapeDtypeStruct((N, D), jnp.float32),
               jax.ShapeDtypeStruct((N, D), jnp.float32)],
)

_tk2 = pl.pallas_call(
    _tk2_body,
    grid=_GRID,
    in_specs=[_row_spec(), _r_spec(), _row_spec(), _full_spec((D, D)),
              _full_spec((D, D)), _full_spec((1, D)), _full_spec((1, D)),
              _degp_spec()],
    out_specs=[_row_spec(), _row_spec()],
    out_shape=[jax.ShapeDtypeStruct((N, D), jnp.float32),
               jax.ShapeDtypeStruct((N, D), jnp.float32)],
)

_tk3 = pl.pallas_call(
    _tk3_body,
    grid=_GRID,
    in_specs=[_r_spec(), _row_spec(), _degp_spec(),
              pl.BlockSpec((BR, G), lambda i: (i, 0)),
              _full_spec((D, 10)), _full_spec((1, 10))],
    out_specs=pl.BlockSpec((G, 10), lambda i: (0, 0)),
    out_shape=jax.ShapeDtypeStruct((G, 10), jnp.float32),
    scratch_shapes=[pltpu.VMEM((G, D), jnp.float32),
                    pltpu.VMEM((G, D), jnp.float32),
                    pltpu.VMEM((G, D), jnp.float32)],
)


def kernel(x, edge_index, edge_attr, batch,
           c0_pre0_W, c0_pre0_b, c0_pre1_W, c0_pre1_b,
           c0_g0_W, c0_g0_b, c0_g1_W, c0_g1_b,
           c1_pre0_W, c1_pre0_b, c1_pre1_W, c1_pre1_b,
           c1_g0_W, c1_g0_b, c1_g1_W, c1_g1_b,
           cls_W, cls_b):
    f32 = jnp.float32
    e = edge_index.shape[1]
    pt = e // NW
    cpt = -(-pt // CH)
    ptp = cpt * CH

    src = edge_index[0].reshape(NW, pt)
    dst = edge_index[1].reshape(NW, pt)
    pad = ptp - pt
    src_pad = jnp.concatenate(
        [src, jnp.zeros((NW, pad), jnp.int32)],
        axis=1).reshape(NW * cpt, 1, CH)
    dst_pad = jnp.concatenate(
        [dst, jnp.full((NW, pad), N, jnp.int32)],
        axis=1).reshape(NW * cpt, 1, CH)

    def mm(a, b):
        return jnp.dot(a, b, preferred_element_type=f32, precision=PREC)

    w_a = 2.0 * mm(c0_pre0_W, c0_g0_W) + mm(c0_pre1_W, c0_g1_W)
    bpre_a = 2.0 * mm(c0_pre0_b[None], c0_g0_W) + mm(c0_pre1_b[None], c0_g1_W)
    bias_a = (2.0 * c0_g0_b + c0_g1_b)[None]
    w_b1 = 2.0 * mm(c1_pre0_W, c1_g0_W)
    w_b2 = mm(c1_pre1_W, c1_g1_W)
    bpre_b = 2.0 * mm(c1_pre0_b[None], c1_g0_W) + mm(c1_pre1_b[None], c1_g1_W)
    bias_b = (2.0 * c1_g0_b + c1_g1_b)[None]

    ones128 = jnp.ones((CH, D), f32)
    zeros128 = jnp.zeros((RPS, D), f32)

    sc_deg = _make_sc_deg(cpt)
    sc_apply = _make_sc_apply(cpt)

    degp = sc_deg(dst_pad, ones128, zeros128)
    mp_a, base_a = _tk1(x, w_a, bpre_a, bias_a, degp)
    r_a = sc_apply(mp_a, src_pad, dst_pad, zeros128)
    mp_b, base_b = _tk2(x, r_a, base_a, w_b1, w_b2, bpre_b, bias_b, degp)
    r_b = sc_apply(mp_b, src_pad, dst_pad, zeros128)

    oneh = (batch[:, None] == jnp.arange(G, dtype=batch.dtype)).astype(f32)
    scores = _tk3(r_b, base_b, degp, oneh, cls_W, cls_b[None])
    return scores


# speedup vs baseline: 24.9916x; 1.2419x over previous
import functools

import jax
import jax.numpy as jnp
from jax import lax
from jax.experimental import pallas as pl
from jax.experimental.pallas import tpu as pltpu
from jax.experimental.pallas import tpu_sc as plsc

N = 10000
D = 128
G = 16
NC = 2
NS = 16
NW = NC * NS
CH = 112
NPAD = 10112
RPS = NPAD // NS
BR = 1000
PREC = lax.Precision.HIGHEST
RING = 3
RI = 4
DEGQ = 6

_mesh = plsc.VectorSubcoreMesh(core_axis_name="c", subcore_axis_name="s")


def _sc_deg_body(dst_hbm, ones_hbm, zeros_hbm, out_hbm, dstv, onesv, acc, sem):
    c = lax.axis_index("c")
    s = lax.axis_index("s")
    w = c * NS + s
    cpt = dst_hbm.shape[0] // NW
    pltpu.sync_copy(zeros_hbm, acc.at[pl.ds(s * RPS, RPS)])
    pltpu.sync_copy(dst_hbm.at[pl.ds(w * cpt, cpt)], dstv)
    pltpu.sync_copy(ones_hbm, onesv)
    plsc.subcore_barrier()

    def body(j, carry):
        pltpu.async_copy(onesv, acc.at[dstv.at[j, 0]], sem, add=True)

        @pl.when(j >= DEGQ)
        def _():
            pltpu.make_async_copy(ones_hbm, onesv, sem).wait()
        return carry

    lax.fori_loop(0, cpt, body, 0)
    for _ in range(DEGQ):
        pltpu.make_async_copy(ones_hbm, onesv, sem).wait()
    plsc.subcore_barrier()
    pltpu.sync_copy(acc.at[pl.ds(s * RPS, RPS)],
                    out_hbm.at[c, pl.ds(s * RPS, RPS)])


def _sc_apply_body(mp_hbm, src_hbm, dst_hbm, zeros_hbm, out_hbm,
                   srcv, dstv, rowsv, acc, gsem, ssem, isem):
    c = lax.axis_index("c")
    s = lax.axis_index("s")
    w = c * NS + s
    cpt = src_hbm.shape[0] // NW
    base = w * cpt
    pltpu.sync_copy(zeros_hbm, acc.at[pl.ds(s * RPS, RPS)])
    plsc.subcore_barrier()

    def load_idx(j):
        slot = lax.rem(j, RI)
        pltpu.async_copy(src_hbm.at[base + j], srcv.at[slot], isem)
        pltpu.async_copy(dst_hbm.at[base + j], dstv.at[slot], isem)

    def drain_idx_pair():
        pltpu.make_async_copy(src_hbm.at[base], srcv.at[0], isem).wait()
        pltpu.make_async_copy(src_hbm.at[base], dstv.at[0], isem).wait()

    def gather(j):
        pltpu.async_copy(mp_hbm.at[srcv.at[lax.rem(j, RI), 0]],
                         rowsv.at[lax.rem(j, RING)], gsem)

    def wait_gather():
        pltpu.make_async_copy(mp_hbm.at[pl.ds(0, CH)],
                              rowsv.at[0], gsem).wait()

    def scatter(j):
        pltpu.async_copy(rowsv.at[lax.rem(j, RING)],
                         acc.at[dstv.at[lax.rem(j, RI), 0]], ssem, add=True)

    def drain_scatter():
        pltpu.make_async_copy(mp_hbm.at[pl.ds(0, CH)],
                              rowsv.at[0], ssem).wait()

    load_idx(0)
    load_idx(1)
    load_idx(2)
    drain_idx_pair()
    gather(0)
    drain_idx_pair()
    gather(1)

    def body(j, carry):
        wait_gather()
        scatter(j)

        @pl.when(j >= 1)
        def _():
            drain_scatter()

        @pl.when(j + 3 < cpt)
        def _():
            load_idx(j + 3)

        @pl.when(j + 2 < cpt)
        def _():
            drain_idx_pair()
            gather(j + 2)
        return carry

    lax.fori_loop(0, cpt, body, 0)
    drain_scatter()
    plsc.subcore_barrier()
    pltpu.sync_copy(acc.at[pl.ds(s * RPS, RPS)],
                    out_hbm.at[c, pl.ds(s * RPS, RPS)])


def _make_sc_deg(cpt):
    return functools.partial(
        pl.kernel,
        out_type=jax.ShapeDtypeStruct((NC, NPAD, D), jnp.float32),
        mesh=_mesh,
        scratch_types=[
            pltpu.VMEM((cpt, 1, CH), jnp.int32),
            pltpu.VMEM((CH, D), jnp.float32),
            pltpu.VMEM_SHARED((NPAD, D), jnp.float32),
            pltpu.SemaphoreType.DMA,
        ],
    )(_sc_deg_body)


def _make_sc_apply(cpt):
    return functools.partial(
        pl.kernel,
        out_type=jax.ShapeDtypeStruct((NC, NPAD, D), jnp.float32),
        mesh=_mesh,
        scratch_types=[
            pltpu.VMEM((RI, 1, CH), jnp.int32),
            pltpu.VMEM((RI, 1, CH), jnp.int32),
            pltpu.VMEM((RING, CH, D), jnp.float32),
            pltpu.VMEM_SHARED((NPAD, D), jnp.float32),
            pltpu.SemaphoreType.DMA,
            pltpu.SemaphoreType.DMA,
            pltpu.SemaphoreType.DMA,
        ],
    )(_sc_apply_body)


def _dinv_from(degp_ref):
    degsum = degp_ref[0, :, 0:1] + degp_ref[1, :, 0:1]
    deg = degsum + 1.0
    return lax.rsqrt(jnp.maximum(deg, 1.0))


def _tk1_body(x_ref, wa_ref, bpre_ref, bias_ref, degp_ref, mp_ref, base_ref):
    dinv = _dinv_from(degp_ref)
    mu = jnp.dot(x_ref[...], wa_ref[...],
                 preferred_element_type=jnp.float32, precision=PREC)
    mu = mu + bpre_ref[...]
    mp_ref[...] = dinv * mu
    base_ref[...] = (dinv * dinv) * mu + bias_ref[...]


def _tk2_body(x_ref, r_ref, basea_ref, wb1_ref, wb2_ref, bpre_ref, bias_ref,
              degp_ref, mp_ref, base_ref):
    dinv = _dinv_from(degp_ref)
    a = dinv * (r_ref[0] + r_ref[1]) + basea_ref[...]
    mu = (jnp.dot(x_ref[...], wb1_ref[...],
                  preferred_element_type=jnp.float32, precision=PREC)
          + jnp.dot(a, wb2_ref[...],
                    preferred_element_type=jnp.float32, precision=PREC)
          + bpre_ref[...])
    mp_ref[...] = dinv * mu
    base_ref[...] = (dinv * dinv) * mu + bias_ref[...]


def _tk3_body(r_ref, baseb_ref, degp_ref, oneh_ref, clsw_ref, clsb_ref,
              out_ref, sums, maxs, cnts):
    i = pl.program_id(0)

    @pl.when(i == 0)
    def _init():
        sums[...] = jnp.zeros_like(sums)
        maxs[...] = jnp.full_like(maxs, -jnp.inf)
        cnts[...] = jnp.zeros_like(cnts)

    dinv = _dinv_from(degp_ref)
    b = dinv * (r_ref[0] + r_ref[1]) + baseb_ref[...]
    oh = oneh_ref[...]
    dn = (((0,), (0,)), ((), ()))
    sums[...] += lax.dot_general(oh, b, dn,
                                 preferred_element_type=jnp.float32,
                                 precision=PREC)
    cnts[...] += lax.dot_general(oh, jnp.ones_like(b), dn,
                                 preferred_element_type=jnp.float32,
                                 precision=PREC)
    for g in range(G):
        col = oh[:, g:g + 1]
        masked = jnp.where(col > 0.5, b, -jnp.inf)
        mg = jnp.max(masked, axis=0, keepdims=True)
        maxs[g:g + 1, :] = jnp.maximum(maxs[g:g + 1, :], mg)

    @pl.when(i == pl.num_programs(0) - 1)
    def _fin():
        mean = sums[...] / jnp.maximum(cnts[...], 1.0)
        pooled = mean + maxs[...]
        out_ref[...] = (jnp.dot(pooled, clsw_ref[...],
                                preferred_element_type=jnp.float32,
                                precision=PREC)
                        + clsb_ref[...])


def _row_spec():
    return pl.BlockSpec((BR, D), lambda i: (i, 0))


def _full_spec(shape):
    nd = len(shape)
    return pl.BlockSpec(shape, lambda i, _n=nd: (0,) * _n)


def _degp_spec():
    return pl.BlockSpec((NC, BR, D), lambda i: (0, i, 0))


def _r_spec():
    return pl.BlockSpec((NC, BR, D), lambda i: (0, i, 0))


_GRID = (N // BR,)

_tk1 = pl.pallas_call(
    _tk1_body,
    grid=_GRID,
    in_specs=[_row_spec(), _full_spec((D, D)), _full_spec((1, D)),
              _full_spec((1, D)), _degp_spec()],
    out_specs=[_row_spec(), _row_spec()],
    out_shape=[jax.ShapeDtypeStruct((N, D), jnp.float32),
               jax.ShapeDtypeStruct((N, D), jnp.float32)],
)

_tk2 = pl.pallas_call(
    _tk2_body,
    grid=_GRID,
    in_specs=[_row_spec(), _r_spec(), _row_spec(), _full_spec((D, D)),
              _full_spec((D, D)), _full_spec((1, D)), _full_spec((1, D)),
              _degp_spec()],
    out_specs=[_row_spec(), _row_spec()],
    out_shape=[jax.ShapeDtypeStruct((N, D), jnp.float32),
               jax.ShapeDtypeStruct((N, D), jnp.float32)],
)

_tk3 = pl.pallas_call(
    _tk3_body,
    grid=_GRID,
    in_specs=[_r_spec(), _row_spec(), _degp_spec(),
              pl.BlockSpec((BR, G), lambda i: (i, 0)),
              _full_spec((D, 10)), _full_spec((1, 10))],
    out_specs=pl.BlockSpec((G, 10), lambda i: (0, 0)),
    out_shape=jax.ShapeDtypeStruct((G, 10), jnp.float32),
    scratch_shapes=[pltpu.VMEM((G, D), jnp.float32),
                    pltpu.VMEM((G, D), jnp.float32),
                    pltpu.VMEM((G, D), jnp.float32)],
)


def kernel(x, edge_index, edge_attr, batch,
           c0_pre0_W, c0_pre0_b, c0_pre1_W, c0_pre1_b,
           c0_g0_W, c0_g0_b, c0_g1_W, c0_g1_b,
           c1_pre0_W, c1_pre0_b, c1_pre1_W, c1_pre1_b,
           c1_g0_W, c1_g0_b, c1_g1_W, c1_g1_b,
           cls_W, cls_b):
    f32 = jnp.float32
    e = edge_index.shape[1]
    pt = e // NW
    cpt = -(-pt // CH)
    ptp = cpt * CH

    src = edge_index[0].reshape(NW, pt)
    dst = edge_index[1].reshape(NW, pt)
    pad = ptp - pt
    src_pad = jnp.concatenate(
        [src, jnp.zeros((NW, pad), jnp.int32)],
        axis=1).reshape(NW * cpt, 1, CH)
    dst_pad = jnp.concatenate(
        [dst, jnp.full((NW, pad), N, jnp.int32)],
        axis=1).reshape(NW * cpt, 1, CH)

    def mm(a, b):
        return jnp.dot(a, b, preferred_element_type=f32, precision=PREC)

    w_a = 2.0 * mm(c0_pre0_W, c0_g0_W) + mm(c0_pre1_W, c0_g1_W)
    bpre_a = 2.0 * mm(c0_pre0_b[None], c0_g0_W) + mm(c0_pre1_b[None], c0_g1_W)
    bias_a = (2.0 * c0_g0_b + c0_g1_b)[None]
    w_b1 = 2.0 * mm(c1_pre0_W, c1_g0_W)
    w_b2 = mm(c1_pre1_W, c1_g1_W)
    bpre_b = 2.0 * mm(c1_pre0_b[None], c1_g0_W) + mm(c1_pre1_b[None], c1_g1_W)
    bias_b = (2.0 * c1_g0_b + c1_g1_b)[None]

    ones128 = jnp.ones((CH, D), f32)
    zeros128 = jnp.zeros((RPS, D), f32)

    sc_deg = _make_sc_deg(cpt)
    sc_apply = _make_sc_apply(cpt)

    degp = sc_deg(dst_pad, ones128, zeros128)
    mp_a, base_a = _tk1(x, w_a, bpre_a, bias_a, degp)
    r_a = sc_apply(mp_a, src_pad, dst_pad, zeros128)
    mp_b, base_b = _tk2(x, r_a, base_a, w_b1, w_b2, bpre_b, bias_b, degp)
    r_b = sc_apply(mp_b, src_pad, dst_pad, zeros128)

    oneh = (batch[:, None] == jnp.arange(G, dtype=batch.dtype)).astype(f32)
    scores = _tk3(r_b, base_b, degp, oneh, cls_W, cls_b[None])
    return scores


# speedup vs baseline: 25.3879x; 1.0159x over previous
import functools

import jax
import jax.numpy as jnp
from jax import lax
from jax.experimental import pallas as pl
from jax.experimental.pallas import tpu as pltpu
from jax.experimental.pallas import tpu_sc as plsc

N = 10000
D = 128
G = 16
NC = 2
NS = 16
NW = NC * NS
CH = 112
NPAD = 10112
RPS = NPAD // NS
BR = 1000
PREC = lax.Precision.HIGHEST
RING = 3
RI = 4
DEGQ = 6
DEGW = 128

_mesh = plsc.VectorSubcoreMesh(core_axis_name="c", subcore_axis_name="s")


def _sc_deg_body(dst_hbm, ones_hbm, zeros_hbm, out_hbm, dstv, onesv, acc, sem):
    c = lax.axis_index("c")
    s = lax.axis_index("s")
    w = c * NS + s
    cpt = dst_hbm.shape[0] // NW
    pltpu.sync_copy(zeros_hbm, acc.at[pl.ds(s * RPS, RPS)])
    pltpu.sync_copy(dst_hbm.at[pl.ds(w * cpt, cpt)], dstv)
    pltpu.sync_copy(ones_hbm, onesv)
    plsc.subcore_barrier()

    def body(j, carry):
        pltpu.async_copy(onesv, acc.at[dstv.at[j, 0]], sem, add=True)

        @pl.when(j >= DEGQ)
        def _():
            pltpu.make_async_copy(ones_hbm, onesv, sem).wait()
        return carry

    lax.fori_loop(0, cpt, body, 0)
    for _ in range(DEGQ):
        pltpu.make_async_copy(ones_hbm, onesv, sem).wait()
    plsc.subcore_barrier()
    pltpu.sync_copy(acc.at[pl.ds(s * RPS, RPS)],
                    out_hbm.at[c, pl.ds(s * RPS, RPS)])


def _sc_apply_body(mp_hbm, src_hbm, dst_hbm, zeros_hbm, out_hbm,
                   srcv, dstv, rowsv, acc, gsem, ssem, isem):
    c = lax.axis_index("c")
    s = lax.axis_index("s")
    w = c * NS + s
    cpt = src_hbm.shape[0] // NW
    base = w * cpt
    pltpu.sync_copy(zeros_hbm, acc.at[pl.ds(s * RPS, RPS)])
    plsc.subcore_barrier()

    def load_idx(j):
        slot = lax.rem(j, RI)
        pltpu.async_copy(src_hbm.at[base + j], srcv.at[slot], isem)
        pltpu.async_copy(dst_hbm.at[base + j], dstv.at[slot], isem)

    def drain_idx_pair():
        pltpu.make_async_copy(src_hbm.at[base], srcv.at[0], isem).wait()
        pltpu.make_async_copy(src_hbm.at[base], dstv.at[0], isem).wait()

    def gather(j):
        pltpu.async_copy(mp_hbm.at[srcv.at[lax.rem(j, RI), 0]],
                         rowsv.at[lax.rem(j, RING)], gsem)

    def wait_gather():
        pltpu.make_async_copy(mp_hbm.at[pl.ds(0, CH)],
                              rowsv.at[0], gsem).wait()

    def scatter(j):
        pltpu.async_copy(rowsv.at[lax.rem(j, RING)],
                         acc.at[dstv.at[lax.rem(j, RI), 0]], ssem, add=True)

    def drain_scatter():
        pltpu.make_async_copy(mp_hbm.at[pl.ds(0, CH)],
                              rowsv.at[0], ssem).wait()

    load_idx(0)
    load_idx(1)
    load_idx(2)
    drain_idx_pair()
    gather(0)
    drain_idx_pair()
    gather(1)

    def body(j, carry):
        wait_gather()
        scatter(j)

        @pl.when(j >= 1)
        def _():
            drain_scatter()

        @pl.when(j + 3 < cpt)
        def _():
            load_idx(j + 3)

        @pl.when(j + 2 < cpt)
        def _():
            drain_idx_pair()
            gather(j + 2)
        return carry

    lax.fori_loop(0, cpt, body, 0)
    drain_scatter()
    plsc.subcore_barrier()
    pltpu.sync_copy(acc.at[pl.ds(s * RPS, RPS)],
                    out_hbm.at[c, pl.ds(s * RPS, RPS)])


def _make_sc_deg(cpt):
    return functools.partial(
        pl.kernel,
        out_type=jax.ShapeDtypeStruct((NC, NPAD, DEGW), jnp.float32),
        mesh=_mesh,
        scratch_types=[
            pltpu.VMEM((cpt, 1, CH), jnp.int32),
            pltpu.VMEM((CH, DEGW), jnp.float32),
            pltpu.VMEM_SHARED((NPAD, DEGW), jnp.float32),
            pltpu.SemaphoreType.DMA,
        ],
    )(_sc_deg_body)


def _make_sc_apply(cpt):
    return functools.partial(
        pl.kernel,
        out_type=jax.ShapeDtypeStruct((NC, NPAD, D), jnp.float32),
        mesh=_mesh,
        scratch_types=[
            pltpu.VMEM((RI, 1, CH), jnp.int32),
            pltpu.VMEM((RI, 1, CH), jnp.int32),
            pltpu.VMEM((RING, CH, D), jnp.float32),
            pltpu.VMEM_SHARED((NPAD, D), jnp.float32),
            pltpu.SemaphoreType.DMA,
            pltpu.SemaphoreType.DMA,
            pltpu.SemaphoreType.DMA,
        ],
    )(_sc_apply_body)


def _dinv_from(degp_ref):
    degsum = degp_ref[0, :, 0:1] + degp_ref[1, :, 0:1]
    deg = degsum + 1.0
    return lax.rsqrt(jnp.maximum(deg, 1.0))


def _mm_body(x_ref, w_ref, b_ref, out_ref):
    out_ref[...] = jnp.dot(x_ref[...], w_ref[...],
                           preferred_element_type=jnp.float32,
                           precision=PREC) + b_ref[...]


def _tk1_body(mu_ref, degp_ref, bias_ref, mp_ref, base_ref):
    dinv = _dinv_from(degp_ref)
    mu = mu_ref[...]
    mp_ref[...] = dinv * mu
    base_ref[...] = (dinv * dinv) * mu + bias_ref[...]


def _tk2_body(mu1_ref, r_ref, basea_ref, wb2_ref, bias_ref,
              degp_ref, mp_ref, base_ref):
    dinv = _dinv_from(degp_ref)
    a = dinv * (r_ref[0] + r_ref[1]) + basea_ref[...]
    mu = (mu1_ref[...]
          + jnp.dot(a, wb2_ref[...],
                    preferred_element_type=jnp.float32, precision=PREC))
    mp_ref[...] = dinv * mu
    base_ref[...] = (dinv * dinv) * mu + bias_ref[...]


def _tk3_body(r_ref, baseb_ref, degp_ref, oneh_ref, clsw_ref, clsb_ref,
              out_ref, sums, maxs, cnts):
    i = pl.program_id(0)

    @pl.when(i == 0)
    def _init():
        sums[...] = jnp.zeros_like(sums)
        maxs[...] = jnp.full_like(maxs, -jnp.inf)
        cnts[...] = jnp.zeros_like(cnts)

    dinv = _dinv_from(degp_ref)
    b = dinv * (r_ref[0] + r_ref[1]) + baseb_ref[...]
    oh = oneh_ref[...]
    dn = (((0,), (0,)), ((), ()))
    sums[...] += lax.dot_general(oh, b, dn,
                                 preferred_element_type=jnp.float32,
                                 precision=PREC)
    cnts[...] += lax.dot_general(oh, jnp.ones_like(b), dn,
                                 preferred_element_type=jnp.float32,
                                 precision=PREC)
    for g in range(G):
        col = oh[:, g:g + 1]
        masked = jnp.where(col > 0.5, b, -jnp.inf)
        mg = jnp.max(masked, axis=0, keepdims=True)
        maxs[g:g + 1, :] = jnp.maximum(maxs[g:g + 1, :], mg)

    @pl.when(i == pl.num_programs(0) - 1)
    def _fin():
        mean = sums[...] / jnp.maximum(cnts[...], 1.0)
        pooled = mean + maxs[...]
        out_ref[...] = (jnp.dot(pooled, clsw_ref[...],
                                preferred_element_type=jnp.float32,
                                precision=PREC)
                        + clsb_ref[...])


def _row_spec():
    return pl.BlockSpec((BR, D), lambda i: (i, 0))


def _full_spec(shape):
    nd = len(shape)
    return pl.BlockSpec(shape, lambda i, _n=nd: (0,) * _n)


def _degp_spec():
    return pl.BlockSpec((NC, BR, DEGW), lambda i: (0, i, 0))


def _r_spec():
    return pl.BlockSpec((NC, BR, D), lambda i: (0, i, 0))


_GRID = (N // BR,)

_mmk = pl.pallas_call(
    _mm_body,
    grid=_GRID,
    in_specs=[_row_spec(), _full_spec((D, D)), _full_spec((1, D))],
    out_specs=_row_spec(),
    out_shape=jax.ShapeDtypeStruct((N, D), jnp.float32),
)

_tk1 = pl.pallas_call(
    _tk1_body,
    grid=_GRID,
    in_specs=[_row_spec(), _degp_spec(), _full_spec((1, D))],
    out_specs=[_row_spec(), _row_spec()],
    out_shape=[jax.ShapeDtypeStruct((N, D), jnp.float32),
               jax.ShapeDtypeStruct((N, D), jnp.float32)],
)

_tk2 = pl.pallas_call(
    _tk2_body,
    grid=_GRID,
    in_specs=[_row_spec(), _r_spec(), _row_spec(), _full_spec((D, D)),
              _full_spec((1, D)), _degp_spec()],
    out_specs=[_row_spec(), _row_spec()],
    out_shape=[jax.ShapeDtypeStruct((N, D), jnp.float32),
               jax.ShapeDtypeStruct((N, D), jnp.float32)],
)

_tk3 = pl.pallas_call(
    _tk3_body,
    grid=_GRID,
    in_specs=[_r_spec(), _row_spec(), _degp_spec(),
              pl.BlockSpec((BR, G), lambda i: (i, 0)),
              _full_spec((D, 10)), _full_spec((1, 10))],
    out_specs=pl.BlockSpec((G, 10), lambda i: (0, 0)),
    out_shape=jax.ShapeDtypeStruct((G, 10), jnp.float32),
    scratch_shapes=[pltpu.VMEM((G, D), jnp.float32),
                    pltpu.VMEM((G, D), jnp.float32),
                    pltpu.VMEM((G, D), jnp.float32)],
)


def kernel(x, edge_index, edge_attr, batch,
           c0_pre0_W, c0_pre0_b, c0_pre1_W, c0_pre1_b,
           c0_g0_W, c0_g0_b, c0_g1_W, c0_g1_b,
           c1_pre0_W, c1_pre0_b, c1_pre1_W, c1_pre1_b,
           c1_g0_W, c1_g0_b, c1_g1_W, c1_g1_b,
           cls_W, cls_b):
    f32 = jnp.float32
    e = edge_index.shape[1]
    pt = e // NW
    cpt = -(-pt // CH)
    ptp = cpt * CH

    src = edge_index[0].reshape(NW, pt)
    dst = edge_index[1].reshape(NW, pt)
    pad = ptp - pt
    src_pad = jnp.concatenate(
        [src, jnp.zeros((NW, pad), jnp.int32)],
        axis=1).reshape(NW * cpt, 1, CH)
    dst_pad = jnp.concatenate(
        [dst, jnp.full((NW, pad), N, jnp.int32)],
        axis=1).reshape(NW * cpt, 1, CH)

    def mm(a, b):
        return jnp.dot(a, b, preferred_element_type=f32, precision=PREC)

    w_a = 2.0 * mm(c0_pre0_W, c0_g0_W) + mm(c0_pre1_W, c0_g1_W)
    bpre_a = 2.0 * mm(c0_pre0_b[None], c0_g0_W) + mm(c0_pre1_b[None], c0_g1_W)
    bias_a = (2.0 * c0_g0_b + c0_g1_b)[None]
    w_b1 = 2.0 * mm(c1_pre0_W, c1_g0_W)
    w_b2 = mm(c1_pre1_W, c1_g1_W)
    bpre_b = 2.0 * mm(c1_pre0_b[None], c1_g0_W) + mm(c1_pre1_b[None], c1_g1_W)
    bias_b = (2.0 * c1_g0_b + c1_g1_b)[None]

    onesw = jnp.ones((CH, DEGW), f32)
    zerosw = jnp.zeros((RPS, DEGW), f32)
    zeros128 = jnp.zeros((RPS, D), f32)

    sc_deg = _make_sc_deg(cpt)
    sc_apply = _make_sc_apply(cpt)

    degp = sc_deg(dst_pad, onesw, zerosw)
    mu_a = _mmk(x, w_a, bpre_a)
    mu_b1 = _mmk(x, w_b1, bpre_b)
    mp_a, base_a = _tk1(mu_a, degp, bias_a)
    r_a = sc_apply(mp_a, src_pad, dst_pad, zeros128)
    mp_b, base_b = _tk2(mu_b1, r_a, base_a, w_b2, bias_b, degp)
    r_b = sc_apply(mp_b, src_pad, dst_pad, zeros128)

    oneh = (batch[:, None] == jnp.arange(G, dtype=batch.dtype)).astype(f32)
    scores = _tk3(r_b, base_b, degp, oneh, cls_W, cls_b[None])
    return scores
